# Initial kernel scaffold; baseline (speedup 1.0000x reference)
import jax
import jax.numpy as jnp
from jax.experimental import pallas as pl


def kernel(full_target_img, full_prior_img, prior_mask, coords):
    raise NotImplementedError("write your pallas kernel here")


# speedup vs baseline: 1.3091x; 1.3091x over previous
import jax
import jax.numpy as jnp
from jax.experimental import pallas as pl
from jax.experimental.pallas import tpu as pltpu

_TX = 8


def _body(coords_ref, target_ref, prior_ref, mask_ref, out_ref):
    b = pl.program_id(0)
    xt = pl.program_id(1)
    cx = coords_ref[b, 0]
    cy = coords_ref[b, 1]
    cz = coords_ref[b, 2]

    shape = target_ref.shape[2:]
    x_ids = jax.lax.broadcasted_iota(jnp.int32, shape, 0) + xt * _TX
    y_ids = jax.lax.broadcasted_iota(jnp.int32, shape, 1)
    z_ids = jax.lax.broadcasted_iota(jnp.int32, shape, 2)
    m = (x_ids == cx) | (y_ids == cy) | (z_ids == cz)

    zero = jnp.zeros(shape, dtype=out_ref.dtype)
    out_ref[0, 0] = jnp.where(m, target_ref[0, 0], zero)
    out_ref[0, 1] = jnp.where(m, prior_ref[0, 0], zero)
    out_ref[0, 2] = mask_ref[0, 0]


def kernel(full_target_img, full_prior_img, prior_mask, coords):
    B, C, W, H, D = full_target_img.shape
    nxt = W // _TX

    def in_map(b, xt, coords_ref):
        return (b, 0, xt, 0, 0)

    def out_map(b, xt, coords_ref):
        return (b, 0, xt, 0, 0)

    grid_spec = pltpu.PrefetchScalarGridSpec(
        num_scalar_prefetch=1,
        grid=(B, nxt),
        in_specs=[
            pl.BlockSpec((1, 1, _TX, H, D), in_map),
            pl.BlockSpec((1, 1, _TX, H, D), in_map),
            pl.BlockSpec((1, 1, _TX, H, D), in_map),
        ],
        out_specs=pl.BlockSpec((1, 3, _TX, H, D), out_map),
    )

    return pl.pallas_call(
        _body,
        grid_spec=grid_spec,
        out_shape=jax.ShapeDtypeStruct((B, 3, W, H, D), full_target_img.dtype),
    )(coords, full_target_img, full_prior_img, prior_mask)
